# Initial kernel scaffold; baseline (speedup 1.0000x reference)
import jax
import jax.numpy as jnp
from jax.experimental import pallas as pl


def kernel(x, edge_index, batch, W1, b1, W2, b2, Wfc, bfc):
    raise NotImplementedError("write your pallas kernel here")


# speedup vs baseline: 38.2474x; 38.2474x over previous
import functools

import jax
import jax.numpy as jnp
from jax import lax
from jax.experimental import pallas as pl
from jax.experimental.pallas import tpu as pltpu
from jax.experimental.pallas import tpu_sc as plsc

NC = 2
NS = 16
NW = NC * NS
LANES = 128
SUB = 8
CHUNK = LANES * SUB
ROWCH = 800
NUM_GRAPHS = 64
BLK = 2000


def _sc_mesh():
    return plsc.VectorSubcoreMesh(core_axis_name="c", subcore_axis_name="s",
                                  num_cores=NC, num_subcores=NS)


@functools.cache
def _degree_call(e_rows, n_nodes):
    n_chunks = e_rows // SUB
    nz = n_nodes // ROWCH

    @functools.partial(
        pl.kernel,
        out_type=jax.ShapeDtypeStruct((NC * n_nodes,), jnp.float32),
        mesh=_sc_mesh(),
        scratch_types=[
            pltpu.VMEM((SUB, LANES), jnp.int32),
            pltpu.VMEM((LANES,), jnp.float32),
            pltpu.VMEM((ROWCH,), jnp.float32),
            pltpu.VMEM_SHARED((n_nodes,), jnp.float32),
        ],
        compiler_params=pltpu.CompilerParams(use_tc_tiling_on_sc=False),
    )
    def deg_kernel(dst_hbm, out_hbm, idx_v, ones_v, stage_v, acc):
        c = lax.axis_index("c")
        s = lax.axis_index("s")
        wid = c * NS + s
        for i in range(LANES // 16):
            ones_v[pl.ds(i * 16, 16)] = jnp.ones((16,), jnp.float32)
        for i in range(ROWCH // 16):
            stage_v[pl.ds(i * 16, 16)] = jnp.zeros((16,), jnp.float32)

        @pl.loop(s, nz, step=NS)
        def _zero(k):
            pltpu.sync_copy(stage_v, acc.at[pl.ds(k * ROWCH, ROWCH)])

        plsc.subcore_barrier()

        @pl.loop(wid, n_chunks, step=NW)
        def _hist(k):
            pltpu.sync_copy(dst_hbm.at[pl.ds(k * SUB, SUB)], idx_v)
            for j in range(SUB):
                pltpu.sync_copy(ones_v, acc.at[idx_v.at[j]], add=True)

        plsc.subcore_barrier()

        @pl.loop(s, nz, step=NS)
        def _flush(k):
            pltpu.sync_copy(acc.at[pl.ds(k * ROWCH, ROWCH)], stage_v)
            pltpu.sync_copy(stage_v,
                            out_hbm.at[pl.ds(c * n_nodes + k * ROWCH, ROWCH)])

    return deg_kernel


@functools.cache
def _scatter_call(n_nodes, h, e_rows):
    n_chunks = e_rows // SUB
    nz = n_nodes // ROWCH

    @functools.partial(
        pl.kernel,
        out_type=jax.ShapeDtypeStruct((NC, n_nodes, h), jnp.float32),
        mesh=_sc_mesh(),
        scratch_types=[
            pltpu.VMEM((SUB, LANES), jnp.int32),
            pltpu.VMEM((SUB, LANES), jnp.int32),
            pltpu.VMEM((CHUNK, h), jnp.float32),
            pltpu.VMEM_SHARED((n_nodes, h), jnp.float32),
            pltpu.SemaphoreType.DMA,
        ],
        compiler_params=pltpu.CompilerParams(use_tc_tiling_on_sc=False),
    )
    def scat_kernel(table_hbm, src_hbm, dst_hbm, out_hbm,
                    src_v, dst_v, rows_v, acc, sem):
        c = lax.axis_index("c")
        s = lax.axis_index("s")
        wid = c * NS + s

        @pl.loop(0, ROWCH)
        def _zrow(i):
            rows_v[i] = jnp.zeros((h,), jnp.float32)

        @pl.loop(s, nz, step=NS)
        def _zero(k):
            pltpu.sync_copy(rows_v.at[pl.ds(0, ROWCH)],
                            acc.at[pl.ds(k * ROWCH, ROWCH)])

        plsc.subcore_barrier()

        @pl.loop(wid, n_chunks, step=NW)
        def _edges(k):
            pltpu.sync_copy(src_hbm.at[pl.ds(k * SUB, SUB)], src_v)
            pltpu.sync_copy(dst_hbm.at[pl.ds(k * SUB, SUB)], dst_v)
            cps = [
                pltpu.async_copy(table_hbm.at[src_v.at[j]],
                                 rows_v.at[pl.ds(j * LANES, LANES)], sem)
                for j in range(SUB)
            ]
            for cp in cps:
                cp.wait()
            for j in range(SUB):
                pltpu.sync_copy(rows_v.at[pl.ds(j * LANES, LANES)],
                                acc.at[dst_v.at[j]], add=True)

        plsc.subcore_barrier()

        @pl.loop(s, nz, step=NS)
        def _flush(k):
            pltpu.sync_copy(acc.at[pl.ds(k * ROWCH, ROWCH)],
                            rows_v.at[pl.ds(0, ROWCH)])
            pltpu.sync_copy(rows_v.at[pl.ds(0, ROWCH)],
                            out_hbm.at[c, pl.ds(k * ROWCH, ROWCH)])

    return scat_kernel


@functools.cache
def _prep1_call(n, f_in, h1):
    def body(x_ref, d0_ref, d1_ref, w1_ref, hs1_ref, dis_ref):
        deg = d0_ref[...] + d1_ref[...] + 1.0
        dis = lax.rsqrt(deg)
        hh = jnp.dot(x_ref[...], w1_ref[...], preferred_element_type=jnp.float32)
        hs1_ref[...] = hh * dis
        dis_ref[...] = dis

    return pl.pallas_call(
        body,
        grid=(n // BLK,),
        in_specs=[
            pl.BlockSpec((BLK, f_in), lambda i: (i, 0)),
            pl.BlockSpec((BLK, 1), lambda i: (i, 0)),
            pl.BlockSpec((BLK, 1), lambda i: (i, 0)),
            pl.BlockSpec((f_in, h1), lambda i: (0, 0)),
        ],
        out_specs=[
            pl.BlockSpec((BLK, h1), lambda i: (i, 0)),
            pl.BlockSpec((BLK, 1), lambda i: (i, 0)),
        ],
        out_shape=[
            jax.ShapeDtypeStruct((n, h1), jnp.float32),
            jax.ShapeDtypeStruct((n, 1), jnp.float32),
        ],
    )


@functools.cache
def _prep2_call(n, h1, h2):
    hh = h2 // 2

    def body(a0_ref, a1_ref, hs1_ref, dis_ref, b1_ref, w2_ref,
             hs2a_ref, hs2b_ref):
        dis = dis_ref[...]
        z = jnp.maximum(
            dis * (a0_ref[...] + a1_ref[...] + hs1_ref[...]) + b1_ref[...], 0.0)
        h2v = jnp.dot(z, w2_ref[...], preferred_element_type=jnp.float32) * dis
        hs2a_ref[...] = h2v[:, :hh]
        hs2b_ref[...] = h2v[:, hh:]

    return pl.pallas_call(
        body,
        grid=(n // BLK,),
        in_specs=[
            pl.BlockSpec((BLK, h1), lambda i: (i, 0)),
            pl.BlockSpec((BLK, h1), lambda i: (i, 0)),
            pl.BlockSpec((BLK, h1), lambda i: (i, 0)),
            pl.BlockSpec((BLK, 1), lambda i: (i, 0)),
            pl.BlockSpec((1, h1), lambda i: (0, 0)),
            pl.BlockSpec((h1, h2), lambda i: (0, 0)),
        ],
        out_specs=[
            pl.BlockSpec((BLK, hh), lambda i: (i, 0)),
            pl.BlockSpec((BLK, hh), lambda i: (i, 0)),
        ],
        out_shape=[
            jax.ShapeDtypeStruct((n, hh), jnp.float32),
            jax.ShapeDtypeStruct((n, hh), jnp.float32),
        ],
    )


@functools.cache
def _final_call(n, hh, out_d):
    def body(a2a0, a2a1, a2b0, a2b1, hs2a, hs2b, dis_ref, b2a, b2b,
             batch_ref, wfca, wfcb, bfc, out_ref, pa, pb, cnt):
        i = pl.program_id(0)

        @pl.when(i == 0)
        def _():
            pa[...] = jnp.zeros_like(pa)
            pb[...] = jnp.zeros_like(pb)
            cnt[...] = jnp.zeros_like(cnt)

        dis = dis_ref[...]
        qa = jnp.maximum(
            dis * (a2a0[...] + a2a1[...] + hs2a[...]) + b2a[...], 0.0)
        qb = jnp.maximum(
            dis * (a2b0[...] + a2b1[...] + hs2b[...]) + b2b[...], 0.0)
        gids = lax.broadcasted_iota(jnp.int32, (NUM_GRAPHS, BLK), 0)
        mask = (gids == batch_ref[0]).astype(jnp.float32)
        pa[...] += jnp.dot(mask, qa, preferred_element_type=jnp.float32)
        pb[...] += jnp.dot(mask, qb, preferred_element_type=jnp.float32)
        cnt[...] += jnp.sum(mask, axis=1, keepdims=True)

        @pl.when(i == pl.num_programs(0) - 1)
        def _():
            c = jnp.maximum(cnt[...], 1.0)
            out_ref[...] = (
                jnp.dot(pa[...] / c, wfca[...], preferred_element_type=jnp.float32)
                + jnp.dot(pb[...] / c, wfcb[...], preferred_element_type=jnp.float32)
                + bfc[...])

    return pl.pallas_call(
        body,
        grid=(n // BLK,),
        in_specs=[
            pl.BlockSpec((BLK, hh), lambda i: (i, 0)),
            pl.BlockSpec((BLK, hh), lambda i: (i, 0)),
            pl.BlockSpec((BLK, hh), lambda i: (i, 0)),
            pl.BlockSpec((BLK, hh), lambda i: (i, 0)),
            pl.BlockSpec((BLK, hh), lambda i: (i, 0)),
            pl.BlockSpec((BLK, hh), lambda i: (i, 0)),
            pl.BlockSpec((BLK, 1), lambda i: (i, 0)),
            pl.BlockSpec((1, hh), lambda i: (0, 0)),
            pl.BlockSpec((1, hh), lambda i: (0, 0)),
            pl.BlockSpec((1, 1, BLK), lambda i: (i, 0, 0)),
            pl.BlockSpec((hh, out_d), lambda i: (0, 0)),
            pl.BlockSpec((hh, out_d), lambda i: (0, 0)),
            pl.BlockSpec((1, out_d), lambda i: (0, 0)),
        ],
        out_specs=pl.BlockSpec((NUM_GRAPHS, out_d), lambda i: (0, 0)),
        out_shape=jax.ShapeDtypeStruct((NUM_GRAPHS, out_d), jnp.float32),
        scratch_shapes=[
            pltpu.VMEM((NUM_GRAPHS, hh), jnp.float32),
            pltpu.VMEM((NUM_GRAPHS, hh), jnp.float32),
            pltpu.VMEM((NUM_GRAPHS, 1), jnp.float32),
        ],
    )


def kernel(x, edge_index, batch, W1, b1, W2, b2, Wfc, bfc):
    n, f_in = x.shape
    e = edge_index.shape[1]
    h1 = W1.shape[1]
    h2 = W2.shape[1]
    out_d = Wfc.shape[1]
    hh = h2 // 2
    assert e % CHUNK == 0 and n % ROWCH == 0 and n % BLK == 0
    assert h1 == 16 and h2 == 32

    src2d = edge_index[0].reshape(e // LANES, LANES)
    dst2d = edge_index[1].reshape(e // LANES, LANES)

    degp = _degree_call(e // LANES, n)(dst2d).reshape(NC, n)
    d0 = degp[0].reshape(n, 1)
    d1 = degp[1].reshape(n, 1)

    hs1, dis = _prep1_call(n, f_in, h1)(x, d0, d1, W1)

    a1 = _scatter_call(n, h1, e // LANES)(hs1, src2d, dst2d)

    hs2a, hs2b = _prep2_call(n, h1, h2)(
        a1[0], a1[1], hs1, dis, b1.reshape(1, h1), W2)

    a2a = _scatter_call(n, hh, e // LANES)(hs2a, src2d, dst2d)
    a2b = _scatter_call(n, hh, e // LANES)(hs2b, src2d, dst2d)

    return _final_call(n, hh, out_d)(
        a2a[0], a2a[1], a2b[0], a2b[1], hs2a, hs2b, dis,
        b2[:hh].reshape(1, hh), b2[hh:].reshape(1, hh),
        batch.reshape(n // BLK, 1, BLK), Wfc[:hh], Wfc[hh:],
        bfc.reshape(1, out_d))


# speedup vs baseline: 55.0328x; 1.4389x over previous
import functools

import jax
import jax.numpy as jnp
import numpy as np
from jax import lax
from jax.experimental import pallas as pl
from jax.experimental.pallas import tpu as pltpu
from jax.experimental.pallas import tpu_sc as plsc

NC = 2
NS = 16
NW = NC * NS
LANES = 128
SUB = 8
CHUNK = LANES * SUB
ROWCH = 1024
NUM_GRAPHS = 64
RBLK = 1600
NPB = 8 * RBLK


def _sc_mesh():
    return plsc.VectorSubcoreMesh(core_axis_name="c", subcore_axis_name="s",
                                  num_cores=NC, num_subcores=NS)


@functools.cache
def _degree_call(e_rows, n_pad):
    n_chunks = e_rows // SUB
    nz = n_pad // ROWCH

    @functools.partial(
        pl.kernel,
        out_type=jax.ShapeDtypeStruct((NC * n_pad,), jnp.float32),
        mesh=_sc_mesh(),
        scratch_types=[
            pltpu.VMEM((SUB, LANES), jnp.int32),
            pltpu.VMEM((LANES,), jnp.float32),
            pltpu.VMEM((ROWCH,), jnp.float32),
            pltpu.VMEM_SHARED((n_pad,), jnp.float32),
        ],
        compiler_params=pltpu.CompilerParams(use_tc_tiling_on_sc=False),
    )
    def deg_kernel(dst_hbm, out_hbm, idx_v, ones_v, stage_v, acc):
        c = lax.axis_index("c")
        s = lax.axis_index("s")
        wid = c * NS + s
        for i in range(LANES // 16):
            ones_v[pl.ds(i * 16, 16)] = jnp.ones((16,), jnp.float32)
        for i in range(ROWCH // 16):
            stage_v[pl.ds(i * 16, 16)] = jnp.zeros((16,), jnp.float32)

        @pl.loop(s, nz, step=NS)
        def _zero(k):
            pltpu.sync_copy(stage_v, acc.at[pl.ds(k * ROWCH, ROWCH)])

        plsc.subcore_barrier()

        @pl.loop(wid, n_chunks, step=NW)
        def _hist(k):
            pltpu.sync_copy(dst_hbm.at[pl.ds(k * SUB, SUB)], idx_v)
            for j in range(SUB):
                pltpu.sync_copy(ones_v, acc.at[idx_v.at[j]], add=True)

        plsc.subcore_barrier()

        @pl.loop(s, nz, step=NS)
        def _flush(k):
            pltpu.sync_copy(acc.at[pl.ds(k * ROWCH, ROWCH)], stage_v)
            pltpu.sync_copy(stage_v,
                            out_hbm.at[pl.ds(c * n_pad + k * ROWCH, ROWCH)])

    return deg_kernel


@functools.cache
def _scatter_call(n_pad, h, e_rows):
    n_chunks = e_rows // SUB
    nz = n_pad // ROWCH

    @functools.partial(
        pl.kernel,
        out_type=jax.ShapeDtypeStruct((NC, n_pad, h), jnp.float32),
        mesh=_sc_mesh(),
        scratch_types=[
            pltpu.VMEM((SUB, LANES), jnp.int32),
            pltpu.VMEM((SUB, LANES), jnp.int32),
            pltpu.VMEM((CHUNK, h), jnp.float32),
            pltpu.VMEM_SHARED((n_pad, h), jnp.float32),
            pltpu.SemaphoreType.DMA,
        ],
        compiler_params=pltpu.CompilerParams(use_tc_tiling_on_sc=False),
    )
    def scat_kernel(table_hbm, src_hbm, dst_hbm, out_hbm,
                    src_v, dst_v, rows_v, acc, sem):
        c = lax.axis_index("c")
        s = lax.axis_index("s")
        wid = c * NS + s

        @pl.loop(0, ROWCH)
        def _zrow(i):
            rows_v[i] = jnp.zeros((h,), jnp.float32)

        @pl.loop(s, nz, step=NS)
        def _zero(k):
            pltpu.sync_copy(rows_v.at[pl.ds(0, ROWCH)],
                            acc.at[pl.ds(k * ROWCH, ROWCH)])

        plsc.subcore_barrier()

        @pl.loop(wid, n_chunks, step=NW)
        def _edges(k):
            pltpu.sync_copy(src_hbm.at[pl.ds(k * SUB, SUB)], src_v)
            pltpu.sync_copy(dst_hbm.at[pl.ds(k * SUB, SUB)], dst_v)
            cps = [
                pltpu.async_copy(table_hbm.at[src_v.at[j]],
                                 rows_v.at[pl.ds(j * LANES, LANES)], sem)
                for j in range(SUB)
            ]
            for cp in cps:
                cp.wait()
            for j in range(SUB):
                pltpu.sync_copy(rows_v.at[pl.ds(j * LANES, LANES)],
                                acc.at[dst_v.at[j]], add=True)

        plsc.subcore_barrier()

        @pl.loop(s, nz, step=NS)
        def _flush(k):
            pltpu.sync_copy(acc.at[pl.ds(k * ROWCH, ROWCH)],
                            rows_v.at[pl.ds(0, ROWCH)])
            pltpu.sync_copy(rows_v.at[pl.ds(0, ROWCH)],
                            out_hbm.at[c, pl.ds(k * ROWCH, ROWCH)])

    return scat_kernel


@functools.cache
def _prep1_call(nrow):
    grid = nrow // RBLK

    def body(x_ref, dp_ref, dq_ref, exp_ref, k1_ref, hs1_ref, dis_ref):
        deg = dp_ref[0] + dq_ref[0] + 1.0
        dis8 = lax.rsqrt(deg)
        dis = jnp.dot(dis8, exp_ref[...], preferred_element_type=jnp.float32)
        h = jnp.dot(x_ref[...], k1_ref[...], preferred_element_type=jnp.float32)
        hs1_ref[...] = h * dis
        dis_ref[...] = dis

    return pl.pallas_call(
        body,
        grid=(grid,),
        in_specs=[
            pl.BlockSpec((RBLK, 128), lambda i: (i, 0)),
            pl.BlockSpec((1, RBLK, 8), lambda i: (0, i, 0)),
            pl.BlockSpec((1, RBLK, 8), lambda i: (1, i, 0)),
            pl.BlockSpec((8, 128), lambda i: (0, 0)),
            pl.BlockSpec((128, 128), lambda i: (0, 0)),
        ],
        out_specs=[
            pl.BlockSpec((RBLK, 128), lambda i: (i, 0)),
            pl.BlockSpec((RBLK, 128), lambda i: (i, 0)),
        ],
        out_shape=[
            jax.ShapeDtypeStruct((nrow, 128), jnp.float32),
            jax.ShapeDtypeStruct((nrow, 128), jnp.float32),
        ],
    )


@functools.cache
def _prep2_call(nrow):
    grid = nrow // RBLK

    def body(a0_ref, a1_ref, hs1_ref, dis_ref, b1_ref, k2a_ref, k2b_ref,
             hs2a_ref, hs2b_ref):
        dis = dis_ref[...]
        z = jnp.maximum(
            dis * (a0_ref[...] + a1_ref[...] + hs1_ref[...]) + b1_ref[...],
            0.0)
        hs2a_ref[...] = jnp.dot(
            z, k2a_ref[...], preferred_element_type=jnp.float32) * dis
        hs2b_ref[...] = jnp.dot(
            z, k2b_ref[...], preferred_element_type=jnp.float32) * dis

    return pl.pallas_call(
        body,
        grid=(grid,),
        in_specs=[
            pl.BlockSpec((RBLK, 128), lambda i: (i, 0)),
            pl.BlockSpec((RBLK, 128), lambda i, g=grid: (i + g, 0)),
            pl.BlockSpec((RBLK, 128), lambda i: (i, 0)),
            pl.BlockSpec((RBLK, 128), lambda i: (i, 0)),
            pl.BlockSpec((1, 128), lambda i: (0, 0)),
            pl.BlockSpec((128, 128), lambda i: (0, 0)),
            pl.BlockSpec((128, 128), lambda i: (0, 0)),
        ],
        out_specs=[
            pl.BlockSpec((RBLK, 128), lambda i: (i, 0)),
            pl.BlockSpec((RBLK, 128), lambda i: (i, 0)),
        ],
        out_shape=[
            jax.ShapeDtypeStruct((nrow, 128), jnp.float32),
            jax.ShapeDtypeStruct((nrow, 128), jnp.float32),
        ],
    )


@functools.cache
def _final_call(nrow, out_d):
    grid = nrow // RBLK

    def body(a2a0, a2a1, a2b0, a2b1, hs2a, hs2b, dis_ref, b2a, b2b,
             bt_ref, wfca, wfcb, bfc, out_ref, pa, pb, cnt):
        i = pl.program_id(0)

        @pl.when(i == 0)
        def _():
            pa[...] = jnp.zeros_like(pa)
            pb[...] = jnp.zeros_like(pb)
            cnt[...] = jnp.zeros_like(cnt)

        dis = dis_ref[...]
        qa = jnp.maximum(
            dis * (a2a0[...] + a2a1[...] + hs2a[...]) + b2a[...], 0.0)
        qb = jnp.maximum(
            dis * (a2b0[...] + a2b1[...] + hs2b[...]) + b2b[...], 0.0)
        bt = bt_ref[0]
        gids = lax.broadcasted_iota(jnp.int32, (NUM_GRAPHS, RBLK), 0)
        for l in range(8):
            mask = (gids == bt[l:l + 1, :]).astype(jnp.float32)
            xa = jnp.dot(mask, qa, preferred_element_type=jnp.float32)
            xb = jnp.dot(mask, qb, preferred_element_type=jnp.float32)
            pa[...] += xa[:, 16 * l:16 * l + 16]
            pb[...] += xb[:, 16 * l:16 * l + 16]
            cnt[...] += jnp.sum(mask, axis=1, keepdims=True)

        @pl.when(i == pl.num_programs(0) - 1)
        def _():
            c = jnp.maximum(cnt[...], 1.0)
            out_ref[...] = (
                jnp.dot(pa[...] / c, wfca[...],
                        preferred_element_type=jnp.float32)
                + jnp.dot(pb[...] / c, wfcb[...],
                          preferred_element_type=jnp.float32)
                + bfc[...])

    return pl.pallas_call(
        body,
        grid=(grid,),
        in_specs=[
            pl.BlockSpec((RBLK, 128), lambda i: (i, 0)),
            pl.BlockSpec((RBLK, 128), lambda i, g=grid: (i + g, 0)),
            pl.BlockSpec((RBLK, 128), lambda i: (i, 0)),
            pl.BlockSpec((RBLK, 128), lambda i, g=grid: (i + g, 0)),
            pl.BlockSpec((RBLK, 128), lambda i: (i, 0)),
            pl.BlockSpec((RBLK, 128), lambda i: (i, 0)),
            pl.BlockSpec((RBLK, 128), lambda i: (i, 0)),
            pl.BlockSpec((1, 128), lambda i: (0, 0)),
            pl.BlockSpec((1, 128), lambda i: (0, 0)),
            pl.BlockSpec((1, 8, RBLK), lambda i: (i, 0, 0)),
            pl.BlockSpec((16, out_d), lambda i: (0, 0)),
            pl.BlockSpec((16, out_d), lambda i: (0, 0)),
            pl.BlockSpec((1, out_d), lambda i: (0, 0)),
        ],
        out_specs=pl.BlockSpec((NUM_GRAPHS, out_d), lambda i: (0, 0)),
        out_shape=jax.ShapeDtypeStruct((NUM_GRAPHS, out_d), jnp.float32),
        scratch_shapes=[
            pltpu.VMEM((NUM_GRAPHS, 16), jnp.float32),
            pltpu.VMEM((NUM_GRAPHS, 16), jnp.float32),
            pltpu.VMEM((NUM_GRAPHS, 1), jnp.float32),
        ],
    )


def kernel(x, edge_index, batch, W1, b1, W2, b2, Wfc, bfc):
    n, f_in = x.shape
    e = edge_index.shape[1]
    h1 = W1.shape[1]
    h2 = W2.shape[1]
    out_d = Wfc.shape[1]
    hh = h2 // 2
    assert f_in == 16 and h1 == 16 and h2 == 32
    assert e % CHUNK == 0

    n_pad = ((n + NPB - 1) // NPB) * NPB
    nrow = n_pad // 8

    x_p = jnp.pad(x, ((0, n_pad - n), (0, 0))).reshape(nrow, 128)
    batch_p = jnp.pad(batch, (0, n_pad - n),
                      constant_values=NUM_GRAPHS)
    bt = batch_p.reshape(nrow // RBLK, RBLK, 8).transpose(0, 2, 1)
    src2d = edge_index[0].reshape(e // LANES, LANES)
    dst2d = edge_index[1].reshape(e // LANES, LANES)

    eye8 = jnp.eye(8, dtype=jnp.float32)
    k1 = jnp.kron(eye8, W1)
    k2a = jnp.kron(eye8, W2[:, :hh])
    k2b = jnp.kron(eye8, W2[:, hh:])
    expand = jnp.kron(eye8, jnp.ones((1, 16), jnp.float32))
    b1t = jnp.tile(b1, 8).reshape(1, 128)
    b2at = jnp.tile(b2[:hh], 8).reshape(1, 128)
    b2bt = jnp.tile(b2[hh:], 8).reshape(1, 128)

    degp = _degree_call(e // LANES, n_pad)(dst2d).reshape(2, nrow, 8)

    hs1_p, dis_p = _prep1_call(nrow)(x_p, degp, degp, expand, k1)

    a1 = _scatter_call(n_pad, h1, e // LANES)(
        hs1_p.reshape(n_pad, h1), src2d, dst2d).reshape(2 * nrow, 128)

    hs2a_p, hs2b_p = _prep2_call(nrow)(a1, a1, hs1_p, dis_p, b1t, k2a, k2b)

    a2a = _scatter_call(n_pad, hh, e // LANES)(
        hs2a_p.reshape(n_pad, hh), src2d, dst2d).reshape(2 * nrow, 128)
    a2b = _scatter_call(n_pad, hh, e // LANES)(
        hs2b_p.reshape(n_pad, hh), src2d, dst2d).reshape(2 * nrow, 128)

    return _final_call(nrow, out_d)(
        a2a, a2a, a2b, a2b, hs2a_p, hs2b_p, dis_p,
        b2at, b2bt, bt, Wfc[:hh], Wfc[hh:], bfc.reshape(1, out_d))


# speedup vs baseline: 62.8310x; 1.1417x over previous
import functools

import jax
import jax.numpy as jnp
import numpy as np
from jax import lax
from jax.experimental import pallas as pl
from jax.experimental.pallas import tpu as pltpu
from jax.experimental.pallas import tpu_sc as plsc

NC = 2
NS = 16
NW = NC * NS
LANES = 128
SUB = 8
CHUNK = LANES * SUB
ROWCH = 1024
NUM_GRAPHS = 64
RBLK = 1600
NPB = 8 * RBLK


def _sc_mesh():
    return plsc.VectorSubcoreMesh(core_axis_name="c", subcore_axis_name="s",
                                  num_cores=NC, num_subcores=NS)


@functools.cache
def _degree_call(e_rows, n_pad):
    n_chunks = e_rows // SUB
    nz = n_pad // ROWCH

    @functools.partial(
        pl.kernel,
        out_type=jax.ShapeDtypeStruct((NC * n_pad,), jnp.float32),
        mesh=_sc_mesh(),
        scratch_types=[
            pltpu.VMEM((SUB, LANES), jnp.int32),
            pltpu.VMEM((LANES,), jnp.float32),
            pltpu.VMEM((ROWCH,), jnp.float32),
            pltpu.VMEM_SHARED((n_pad,), jnp.float32),
        ],
        compiler_params=pltpu.CompilerParams(use_tc_tiling_on_sc=False),
    )
    def deg_kernel(dst_hbm, out_hbm, idx_v, ones_v, stage_v, acc):
        c = lax.axis_index("c")
        s = lax.axis_index("s")
        wid = c * NS + s
        for i in range(LANES // 16):
            ones_v[pl.ds(i * 16, 16)] = jnp.ones((16,), jnp.float32)
        for i in range(ROWCH // 16):
            stage_v[pl.ds(i * 16, 16)] = jnp.zeros((16,), jnp.float32)

        @pl.loop(s, nz, step=NS)
        def _zero(k):
            pltpu.sync_copy(stage_v, acc.at[pl.ds(k * ROWCH, ROWCH)])

        plsc.subcore_barrier()

        @pl.loop(wid, n_chunks, step=NW)
        def _hist(k):
            pltpu.sync_copy(dst_hbm.at[pl.ds(k * SUB, SUB)], idx_v)
            for j in range(SUB):
                pltpu.sync_copy(ones_v, acc.at[idx_v.at[j]], add=True)

        plsc.subcore_barrier()

        @pl.loop(s, nz, step=NS)
        def _flush(k):
            pltpu.sync_copy(acc.at[pl.ds(k * ROWCH, ROWCH)], stage_v)
            pltpu.sync_copy(stage_v,
                            out_hbm.at[pl.ds(c * n_pad + k * ROWCH, ROWCH)])

    return deg_kernel


@functools.cache
def _scatter_call(n_pad, h, e_rows):
    SUBS = 4
    CHUNKS = SUBS * LANES
    n_chunks = e_rows // SUBS
    nz = n_pad // CHUNKS

    @functools.partial(
        pl.kernel,
        out_type=jax.ShapeDtypeStruct((NC, n_pad, h), jnp.float32),
        mesh=_sc_mesh(),
        scratch_types=[
            pltpu.VMEM((2, SUBS, LANES), jnp.int32),
            pltpu.VMEM((2, SUBS, LANES), jnp.int32),
            pltpu.VMEM((2, CHUNKS, h), jnp.float32),
            pltpu.VMEM_SHARED((n_pad, h), jnp.float32),
            pltpu.SemaphoreType.DMA,
            pltpu.SemaphoreType.DMA,
        ],
        compiler_params=pltpu.CompilerParams(use_tc_tiling_on_sc=False),
    )
    def scat_kernel(table_hbm, src_hbm, dst_hbm, out_hbm,
                    src_v, dst_v, rows_v, acc, sem0, sem1):
        c = lax.axis_index("c")
        s = lax.axis_index("s")
        wid = c * NS + s
        sg = (sem0, sem1)
        ss = (sem0, sem1)
        n_my = (n_chunks + NW - 1 - wid) // NW

        def load_and_gather(b, i):
            k = wid + i * NW
            pltpu.sync_copy(src_hbm.at[pl.ds(k * SUBS, SUBS)], src_v.at[b])
            pltpu.sync_copy(dst_hbm.at[pl.ds(k * SUBS, SUBS)], dst_v.at[b])
            for j in range(SUBS):
                pltpu.async_copy(table_hbm.at[src_v.at[b, j]],
                                 rows_v.at[b].at[pl.ds(j * LANES, LANES)],
                                 sg[b])

        def wait_gathers(b):
            for j in range(SUBS):
                pltpu.make_async_copy(
                    table_hbm.at[src_v.at[b, j]],
                    rows_v.at[b].at[pl.ds(j * LANES, LANES)], sg[b]).wait()

        def fire_scatters(b):
            for j in range(SUBS):
                pltpu.async_copy(rows_v.at[b].at[pl.ds(j * LANES, LANES)],
                                 acc.at[dst_v.at[b, j]], ss[b], add=True)

        def wait_scatters(b):
            for j in range(SUBS):
                pltpu.make_async_copy(
                    rows_v.at[b].at[pl.ds(j * LANES, LANES)],
                    acc.at[dst_v.at[b, j]], ss[b]).wait()

        @pl.loop(0, CHUNKS)
        def _zrow(i):
            rows_v[0, i] = jnp.zeros((h,), jnp.float32)

        @pl.loop(s, nz, step=NS)
        def _zero(k):
            pltpu.sync_copy(rows_v.at[0].at[pl.ds(0, CHUNKS)],
                            acc.at[pl.ds(k * CHUNKS, CHUNKS)])

        plsc.subcore_barrier()

        for b in range(2):
            @pl.when(b < n_my)
            def _prime(b=b):
                load_and_gather(b, jnp.int32(b))

        n_pairs = (n_my + 1) // 2

        @pl.loop(0, n_pairs)
        def _edges(p):
            for b in range(2):
                i = 2 * p + b

                @pl.when(i < n_my)
                def _consume(b=b):
                    wait_gathers(b)
                    fire_scatters(b)
            for b in range(2):
                i_next = 2 * p + 2 + b

                @pl.when(i_next < n_my)
                def _refill(b=b, i_next=i_next):
                    wait_scatters(b)
                    load_and_gather(b, i_next)

        for b in range(2):
            @pl.when(b < n_my)
            def _drain(b=b):
                wait_scatters(b)

        plsc.subcore_barrier()

        @pl.loop(s, nz, step=NS)
        def _flush(k):
            pltpu.sync_copy(acc.at[pl.ds(k * CHUNKS, CHUNKS)],
                            rows_v.at[0].at[pl.ds(0, CHUNKS)])
            pltpu.sync_copy(rows_v.at[0].at[pl.ds(0, CHUNKS)],
                            out_hbm.at[c, pl.ds(k * CHUNKS, CHUNKS)])

    return scat_kernel


@functools.cache
def _prep1_call(nrow):
    grid = nrow // RBLK

    def body(x_ref, dp_ref, dq_ref, exp_ref, k1_ref, hs1_ref, dis_ref):
        deg = dp_ref[0] + dq_ref[0] + 1.0
        dis8 = lax.rsqrt(deg)
        dis = jnp.dot(dis8, exp_ref[...], preferred_element_type=jnp.float32)
        h = jnp.dot(x_ref[...], k1_ref[...], preferred_element_type=jnp.float32)
        hs1_ref[...] = h * dis
        dis_ref[...] = dis

    return pl.pallas_call(
        body,
        grid=(grid,),
        in_specs=[
            pl.BlockSpec((RBLK, 128), lambda i: (i, 0)),
            pl.BlockSpec((1, RBLK, 8), lambda i: (0, i, 0)),
            pl.BlockSpec((1, RBLK, 8), lambda i: (1, i, 0)),
            pl.BlockSpec((8, 128), lambda i: (0, 0)),
            pl.BlockSpec((128, 128), lambda i: (0, 0)),
        ],
        out_specs=[
            pl.BlockSpec((RBLK, 128), lambda i: (i, 0)),
            pl.BlockSpec((RBLK, 128), lambda i: (i, 0)),
        ],
        out_shape=[
            jax.ShapeDtypeStruct((nrow, 128), jnp.float32),
            jax.ShapeDtypeStruct((nrow, 128), jnp.float32),
        ],
    )


@functools.cache
def _prep2_call(nrow):
    grid = nrow // RBLK

    def body(a0_ref, a1_ref, hs1_ref, dis_ref, b1_ref, k2a_ref, k2b_ref,
             hs2a_ref, hs2b_ref):
        dis = dis_ref[...]
        z = jnp.maximum(
            dis * (a0_ref[...] + a1_ref[...] + hs1_ref[...]) + b1_ref[...],
            0.0)
        hs2a_ref[...] = jnp.dot(
            z, k2a_ref[...], preferred_element_type=jnp.float32) * dis
        hs2b_ref[...] = jnp.dot(
            z, k2b_ref[...], preferred_element_type=jnp.float32) * dis

    return pl.pallas_call(
        body,
        grid=(grid,),
        in_specs=[
            pl.BlockSpec((RBLK, 128), lambda i: (i, 0)),
            pl.BlockSpec((RBLK, 128), lambda i, g=grid: (i + g, 0)),
            pl.BlockSpec((RBLK, 128), lambda i: (i, 0)),
            pl.BlockSpec((RBLK, 128), lambda i: (i, 0)),
            pl.BlockSpec((1, 128), lambda i: (0, 0)),
            pl.BlockSpec((128, 128), lambda i: (0, 0)),
            pl.BlockSpec((128, 128), lambda i: (0, 0)),
        ],
        out_specs=[
            pl.BlockSpec((RBLK, 128), lambda i: (i, 0)),
            pl.BlockSpec((RBLK, 128), lambda i: (i, 0)),
        ],
        out_shape=[
            jax.ShapeDtypeStruct((nrow, 128), jnp.float32),
            jax.ShapeDtypeStruct((nrow, 128), jnp.float32),
        ],
    )


@functools.cache
def _final_call(nrow, out_d):
    grid = nrow // RBLK

    def body(a2a0, a2a1, a2b0, a2b1, hs2a, hs2b, dis_ref, b2a, b2b,
             bt_ref, wfca, wfcb, bfc, out_ref, pa, pb, cnt):
        i = pl.program_id(0)

        @pl.when(i == 0)
        def _():
            pa[...] = jnp.zeros_like(pa)
            pb[...] = jnp.zeros_like(pb)
            cnt[...] = jnp.zeros_like(cnt)

        dis = dis_ref[...]
        qa = jnp.maximum(
            dis * (a2a0[...] + a2a1[...] + hs2a[...]) + b2a[...], 0.0)
        qb = jnp.maximum(
            dis * (a2b0[...] + a2b1[...] + hs2b[...]) + b2b[...], 0.0)
        bt = bt_ref[0]
        gids = lax.broadcasted_iota(jnp.int32, (NUM_GRAPHS, RBLK), 0)
        for l in range(8):
            mask = (gids == bt[l:l + 1, :]).astype(jnp.float32)
            xa = jnp.dot(mask, qa, preferred_element_type=jnp.float32)
            xb = jnp.dot(mask, qb, preferred_element_type=jnp.float32)
            pa[...] += xa[:, 16 * l:16 * l + 16]
            pb[...] += xb[:, 16 * l:16 * l + 16]
            cnt[...] += jnp.sum(mask, axis=1, keepdims=True)

        @pl.when(i == pl.num_programs(0) - 1)
        def _():
            c = jnp.maximum(cnt[...], 1.0)
            out_ref[...] = (
                jnp.dot(pa[...] / c, wfca[...],
                        preferred_element_type=jnp.float32)
                + jnp.dot(pb[...] / c, wfcb[...],
                          preferred_element_type=jnp.float32)
                + bfc[...])

    return pl.pallas_call(
        body,
        grid=(grid,),
        in_specs=[
            pl.BlockSpec((RBLK, 128), lambda i: (i, 0)),
            pl.BlockSpec((RBLK, 128), lambda i, g=grid: (i + g, 0)),
            pl.BlockSpec((RBLK, 128), lambda i: (i, 0)),
            pl.BlockSpec((RBLK, 128), lambda i, g=grid: (i + g, 0)),
            pl.BlockSpec((RBLK, 128), lambda i: (i, 0)),
            pl.BlockSpec((RBLK, 128), lambda i: (i, 0)),
            pl.BlockSpec((RBLK, 128), lambda i: (i, 0)),
            pl.BlockSpec((1, 128), lambda i: (0, 0)),
            pl.BlockSpec((1, 128), lambda i: (0, 0)),
            pl.BlockSpec((1, 8, RBLK), lambda i: (i, 0, 0)),
            pl.BlockSpec((16, out_d), lambda i: (0, 0)),
            pl.BlockSpec((16, out_d), lambda i: (0, 0)),
            pl.BlockSpec((1, out_d), lambda i: (0, 0)),
        ],
        out_specs=pl.BlockSpec((NUM_GRAPHS, out_d), lambda i: (0, 0)),
        out_shape=jax.ShapeDtypeStruct((NUM_GRAPHS, out_d), jnp.float32),
        scratch_shapes=[
            pltpu.VMEM((NUM_GRAPHS, 16), jnp.float32),
            pltpu.VMEM((NUM_GRAPHS, 16), jnp.float32),
            pltpu.VMEM((NUM_GRAPHS, 1), jnp.float32),
        ],
    )


def kernel(x, edge_index, batch, W1, b1, W2, b2, Wfc, bfc):
    n, f_in = x.shape
    e = edge_index.shape[1]
    h1 = W1.shape[1]
    h2 = W2.shape[1]
    out_d = Wfc.shape[1]
    hh = h2 // 2
    assert f_in == 16 and h1 == 16 and h2 == 32
    assert e % CHUNK == 0

    n_pad = ((n + NPB - 1) // NPB) * NPB
    nrow = n_pad // 8

    x_p = jnp.pad(x, ((0, n_pad - n), (0, 0))).reshape(nrow, 128)
    batch_p = jnp.pad(batch, (0, n_pad - n),
                      constant_values=NUM_GRAPHS)
    bt = batch_p.reshape(nrow // RBLK, RBLK, 8).transpose(0, 2, 1)
    src2d = edge_index[0].reshape(e // LANES, LANES)
    dst2d = edge_index[1].reshape(e // LANES, LANES)

    eye8 = jnp.eye(8, dtype=jnp.float32)
    k1 = jnp.kron(eye8, W1)
    k2a = jnp.kron(eye8, W2[:, :hh])
    k2b = jnp.kron(eye8, W2[:, hh:])
    expand = jnp.kron(eye8, jnp.ones((1, 16), jnp.float32))
    b1t = jnp.tile(b1, 8).reshape(1, 128)
    b2at = jnp.tile(b2[:hh], 8).reshape(1, 128)
    b2bt = jnp.tile(b2[hh:], 8).reshape(1, 128)

    degp = _degree_call(e // LANES, n_pad)(dst2d).reshape(2, nrow, 8)

    hs1_p, dis_p = _prep1_call(nrow)(x_p, degp, degp, expand, k1)

    a1 = _scatter_call(n_pad, h1, e // LANES)(
        hs1_p.reshape(n_pad, h1), src2d, dst2d).reshape(2 * nrow, 128)

    hs2a_p, hs2b_p = _prep2_call(nrow)(a1, a1, hs1_p, dis_p, b1t, k2a, k2b)

    a2a = _scatter_call(n_pad, hh, e // LANES)(
        hs2a_p.reshape(n_pad, hh), src2d, dst2d).reshape(2 * nrow, 128)
    a2b = _scatter_call(n_pad, hh, e // LANES)(
        hs2b_p.reshape(n_pad, hh), src2d, dst2d).reshape(2 * nrow, 128)

    return _final_call(nrow, out_d)(
        a2a, a2a, a2b, a2b, hs2a_p, hs2b_p, dis_p,
        b2at, b2bt, bt, Wfc[:hh], Wfc[hh:], bfc.reshape(1, out_d))


# speedup vs baseline: 80.1022x; 1.2749x over previous
import functools

import jax
import jax.numpy as jnp
import numpy as np
from jax import lax
from jax.experimental import pallas as pl
from jax.experimental.pallas import tpu as pltpu
from jax.experimental.pallas import tpu_sc as plsc

NC = 2
NS = 16
NW = NC * NS
LANES = 128
SUB = 8
CHUNK = LANES * SUB
ROWCH = 1024
NUM_GRAPHS = 64
RBLK = 1600
NPB = 8 * RBLK


def _sc_mesh():
    return plsc.VectorSubcoreMesh(core_axis_name="c", subcore_axis_name="s",
                                  num_cores=NC, num_subcores=NS)


@functools.cache
def _degree_call(e_rows, n_pad):
    n_chunks = e_rows // SUB
    nz = n_pad // ROWCH

    @functools.partial(
        pl.kernel,
        out_type=jax.ShapeDtypeStruct((NC * n_pad,), jnp.float32),
        mesh=_sc_mesh(),
        scratch_types=[
            pltpu.VMEM((SUB, LANES), jnp.int32),
            pltpu.VMEM((LANES,), jnp.float32),
            pltpu.VMEM((ROWCH,), jnp.float32),
            pltpu.VMEM_SHARED((n_pad,), jnp.float32),
        ],
        compiler_params=pltpu.CompilerParams(use_tc_tiling_on_sc=False),
    )
    def deg_kernel(dst_hbm, out_hbm, idx_v, ones_v, stage_v, acc):
        c = lax.axis_index("c")
        s = lax.axis_index("s")
        wid = c * NS + s
        for i in range(LANES // 16):
            ones_v[pl.ds(i * 16, 16)] = jnp.ones((16,), jnp.float32)
        for i in range(ROWCH // 16):
            stage_v[pl.ds(i * 16, 16)] = jnp.zeros((16,), jnp.float32)

        @pl.loop(s, nz, step=NS)
        def _zero(k):
            pltpu.sync_copy(stage_v, acc.at[pl.ds(k * ROWCH, ROWCH)])

        plsc.subcore_barrier()

        @pl.loop(wid, n_chunks, step=NW)
        def _hist(k):
            pltpu.sync_copy(dst_hbm.at[pl.ds(k * SUB, SUB)], idx_v)
            for j in range(SUB):
                pltpu.sync_copy(ones_v, acc.at[idx_v.at[j]], add=True)

        plsc.subcore_barrier()

        @pl.loop(s, nz, step=NS)
        def _flush(k):
            pltpu.sync_copy(acc.at[pl.ds(k * ROWCH, ROWCH)], stage_v)
            pltpu.sync_copy(stage_v,
                            out_hbm.at[pl.ds(c * n_pad + k * ROWCH, ROWCH)])

    return deg_kernel


@functools.cache
def _scatter_call(n_pad, h, e_rows):
    SUBS = 4
    CHUNKS = SUBS * LANES
    n_chunks = e_rows // SUBS
    nz = n_pad // CHUNKS

    @functools.partial(
        pl.kernel,
        out_type=jax.ShapeDtypeStruct((NC, n_pad, h), jnp.float32),
        mesh=_sc_mesh(),
        scratch_types=[
            pltpu.VMEM((2, SUBS, 2, LANES), jnp.int32),
            pltpu.VMEM((2, CHUNKS, h), jnp.float32),
            pltpu.VMEM_SHARED((n_pad, h), jnp.float32),
            pltpu.SemaphoreType.DMA,
            pltpu.SemaphoreType.DMA,
        ],
        compiler_params=pltpu.CompilerParams(use_tc_tiling_on_sc=False),
    )
    def scat_kernel(table_hbm, ei_hbm, out_hbm,
                    idx_v, rows_v, acc, sem0, sem1):
        c = lax.axis_index("c")
        s = lax.axis_index("s")
        wid = c * NS + s
        sg = (sem0, sem1)
        ss = (sem0, sem1)
        n_my = (n_chunks + NW - 1 - wid) // NW

        def load_and_gather(b, i):
            k = wid + i * NW
            pltpu.sync_copy(ei_hbm.at[pl.ds(k * SUBS, SUBS)], idx_v.at[b])
            for j in range(SUBS):
                pltpu.async_copy(table_hbm.at[idx_v.at[b, j, 0]],
                                 rows_v.at[b].at[pl.ds(j * LANES, LANES)],
                                 sg[b])

        def wait_gathers(b):
            for j in range(SUBS):
                pltpu.make_async_copy(
                    table_hbm.at[idx_v.at[b, j, 0]],
                    rows_v.at[b].at[pl.ds(j * LANES, LANES)], sg[b]).wait()

        def fire_scatters(b):
            for j in range(SUBS):
                pltpu.async_copy(rows_v.at[b].at[pl.ds(j * LANES, LANES)],
                                 acc.at[idx_v.at[b, j, 1]], ss[b], add=True)

        def wait_scatters(b):
            for j in range(SUBS):
                pltpu.make_async_copy(
                    rows_v.at[b].at[pl.ds(j * LANES, LANES)],
                    acc.at[idx_v.at[b, j, 1]], ss[b]).wait()

        @pl.loop(0, CHUNKS)
        def _zrow(i):
            rows_v[0, i] = jnp.zeros((h,), jnp.float32)

        @pl.loop(s, nz, step=NS)
        def _zero(k):
            pltpu.sync_copy(rows_v.at[0].at[pl.ds(0, CHUNKS)],
                            acc.at[pl.ds(k * CHUNKS, CHUNKS)])

        plsc.subcore_barrier()

        for b in range(2):
            @pl.when(b < n_my)
            def _prime(b=b):
                load_and_gather(b, jnp.int32(b))

        n_pairs = (n_my + 1) // 2

        @pl.loop(0, n_pairs)
        def _edges(p):
            for b in range(2):
                i = 2 * p + b

                @pl.when(i < n_my)
                def _consume(b=b):
                    wait_gathers(b)
                    fire_scatters(b)
            for b in range(2):
                i_next = 2 * p + 2 + b

                @pl.when(i_next < n_my)
                def _refill(b=b, i_next=i_next):
                    wait_scatters(b)
                    load_and_gather(b, i_next)

        for b in range(2):
            @pl.when(b < n_my)
            def _drain(b=b):
                wait_scatters(b)

        plsc.subcore_barrier()

        @pl.loop(s, nz, step=NS)
        def _flush(k):
            pltpu.sync_copy(acc.at[pl.ds(k * CHUNKS, CHUNKS)],
                            rows_v.at[0].at[pl.ds(0, CHUNKS)])
            pltpu.sync_copy(rows_v.at[0].at[pl.ds(0, CHUNKS)],
                            out_hbm.at[c, pl.ds(k * CHUNKS, CHUNKS)])

    return scat_kernel


@functools.cache
def _prep1_call(nrow):
    grid = nrow // RBLK

    def body(x_ref, dp_ref, dq_ref, exp_ref, k1_ref, hs1_ref, dis_ref):
        deg = dp_ref[0] + dq_ref[0] + 1.0
        dis8 = lax.rsqrt(deg)
        dis = jnp.dot(dis8, exp_ref[...], preferred_element_type=jnp.float32)
        h = jnp.dot(x_ref[...], k1_ref[...], preferred_element_type=jnp.float32)
        hs1_ref[...] = h * dis
        dis_ref[...] = dis

    return pl.pallas_call(
        body,
        grid=(grid,),
        in_specs=[
            pl.BlockSpec((RBLK, 128), lambda i: (i, 0)),
            pl.BlockSpec((1, RBLK, 8), lambda i: (0, i, 0)),
            pl.BlockSpec((1, RBLK, 8), lambda i: (1, i, 0)),
            pl.BlockSpec((8, 128), lambda i: (0, 0)),
            pl.BlockSpec((128, 128), lambda i: (0, 0)),
        ],
        out_specs=[
            pl.BlockSpec((RBLK, 128), lambda i: (i, 0)),
            pl.BlockSpec((RBLK, 128), lambda i: (i, 0)),
        ],
        out_shape=[
            jax.ShapeDtypeStruct((nrow, 128), jnp.float32),
            jax.ShapeDtypeStruct((nrow, 128), jnp.float32),
        ],
    )


@functools.cache
def _prep2_call(nrow):
    grid = nrow // RBLK

    def body(a0_ref, a1_ref, hs1_ref, dis_ref, b1_ref, k2a_ref, k2b_ref,
             hs2a_ref, hs2b_ref):
        dis = dis_ref[...]
        z = jnp.maximum(
            dis * (a0_ref[...] + a1_ref[...] + hs1_ref[...]) + b1_ref[...],
            0.0)
        hs2a_ref[...] = jnp.dot(
            z, k2a_ref[...], preferred_element_type=jnp.float32) * dis
        hs2b_ref[...] = jnp.dot(
            z, k2b_ref[...], preferred_element_type=jnp.float32) * dis

    return pl.pallas_call(
        body,
        grid=(grid,),
        in_specs=[
            pl.BlockSpec((RBLK, 128), lambda i: (i, 0)),
            pl.BlockSpec((RBLK, 128), lambda i, g=grid: (i + g, 0)),
            pl.BlockSpec((RBLK, 128), lambda i: (i, 0)),
            pl.BlockSpec((RBLK, 128), lambda i: (i, 0)),
            pl.BlockSpec((1, 128), lambda i: (0, 0)),
            pl.BlockSpec((128, 128), lambda i: (0, 0)),
            pl.BlockSpec((128, 128), lambda i: (0, 0)),
        ],
        out_specs=[
            pl.BlockSpec((RBLK, 128), lambda i: (i, 0)),
            pl.BlockSpec((RBLK, 128), lambda i: (i, 0)),
        ],
        out_shape=[
            jax.ShapeDtypeStruct((nrow, 128), jnp.float32),
            jax.ShapeDtypeStruct((nrow, 128), jnp.float32),
        ],
    )


@functools.cache
def _final_call(nrow, out_d):
    grid = nrow // RBLK

    def body(a2a0, a2a1, a2b0, a2b1, hs2a, hs2b, dis_ref, b2a, b2b,
             bt_ref, wfca, wfcb, bfc, out_ref, pa, pb, cnt):
        i = pl.program_id(0)

        @pl.when(i == 0)
        def _():
            pa[...] = jnp.zeros_like(pa)
            pb[...] = jnp.zeros_like(pb)
            cnt[...] = jnp.zeros_like(cnt)

        dis = dis_ref[...]
        qa = jnp.maximum(
            dis * (a2a0[...] + a2a1[...] + hs2a[...]) + b2a[...], 0.0)
        qb = jnp.maximum(
            dis * (a2b0[...] + a2b1[...] + hs2b[...]) + b2b[...], 0.0)
        bt = bt_ref[0]
        gids = lax.broadcasted_iota(jnp.int32, (NUM_GRAPHS, RBLK), 0)
        for l in range(8):
            mask = (gids == bt[l:l + 1, :]).astype(jnp.float32)
            xa = jnp.dot(mask, qa, preferred_element_type=jnp.float32)
            xb = jnp.dot(mask, qb, preferred_element_type=jnp.float32)
            pa[...] += xa[:, 16 * l:16 * l + 16]
            pb[...] += xb[:, 16 * l:16 * l + 16]
            cnt[...] += jnp.sum(mask, axis=1, keepdims=True)

        @pl.when(i == pl.num_programs(0) - 1)
        def _():
            c = jnp.maximum(cnt[...], 1.0)
            out_ref[...] = (
                jnp.dot(pa[...] / c, wfca[...],
                        preferred_element_type=jnp.float32)
                + jnp.dot(pb[...] / c, wfcb[...],
                          preferred_element_type=jnp.float32)
                + bfc[...])

    return pl.pallas_call(
        body,
        grid=(grid,),
        in_specs=[
            pl.BlockSpec((RBLK, 128), lambda i: (i, 0)),
            pl.BlockSpec((RBLK, 128), lambda i, g=grid: (i + g, 0)),
            pl.BlockSpec((RBLK, 128), lambda i: (i, 0)),
            pl.BlockSpec((RBLK, 128), lambda i, g=grid: (i + g, 0)),
            pl.BlockSpec((RBLK, 128), lambda i: (i, 0)),
            pl.BlockSpec((RBLK, 128), lambda i: (i, 0)),
            pl.BlockSpec((RBLK, 128), lambda i: (i, 0)),
            pl.BlockSpec((1, 128), lambda i: (0, 0)),
            pl.BlockSpec((1, 128), lambda i: (0, 0)),
            pl.BlockSpec((1, 8, RBLK), lambda i: (i, 0, 0)),
            pl.BlockSpec((16, out_d), lambda i: (0, 0)),
            pl.BlockSpec((16, out_d), lambda i: (0, 0)),
            pl.BlockSpec((1, out_d), lambda i: (0, 0)),
        ],
        out_specs=pl.BlockSpec((NUM_GRAPHS, out_d), lambda i: (0, 0)),
        out_shape=jax.ShapeDtypeStruct((NUM_GRAPHS, out_d), jnp.float32),
        scratch_shapes=[
            pltpu.VMEM((NUM_GRAPHS, 16), jnp.float32),
            pltpu.VMEM((NUM_GRAPHS, 16), jnp.float32),
            pltpu.VMEM((NUM_GRAPHS, 1), jnp.float32),
        ],
    )


def kernel(x, edge_index, batch, W1, b1, W2, b2, Wfc, bfc):
    n, f_in = x.shape
    e = edge_index.shape[1]
    h1 = W1.shape[1]
    h2 = W2.shape[1]
    out_d = Wfc.shape[1]
    hh = h2 // 2
    assert f_in == 16 and h1 == 16 and h2 == 32
    assert e % CHUNK == 0

    n_pad = ((n + NPB - 1) // NPB) * NPB
    nrow = n_pad // 8

    x_p = jnp.pad(x, ((0, n_pad - n), (0, 0))).reshape(nrow, 128)
    batch_p = jnp.pad(batch, (0, n_pad - n),
                      constant_values=NUM_GRAPHS)
    bt = batch_p.reshape(nrow // RBLK, RBLK, 8).transpose(0, 2, 1)
    dst2d = edge_index[1].reshape(e // LANES, LANES)
    ei2 = jnp.transpose(edge_index.reshape(2, e // LANES, LANES), (1, 0, 2))

    eye8 = jnp.eye(8, dtype=jnp.float32)
    k1 = jnp.kron(eye8, W1)
    k2a = jnp.kron(eye8, W2[:, :hh])
    k2b = jnp.kron(eye8, W2[:, hh:])
    expand = jnp.kron(eye8, jnp.ones((1, 16), jnp.float32))
    b1t = jnp.tile(b1, 8).reshape(1, 128)
    b2at = jnp.tile(b2[:hh], 8).reshape(1, 128)
    b2bt = jnp.tile(b2[hh:], 8).reshape(1, 128)

    degp = _degree_call(e // LANES, n_pad)(dst2d).reshape(2, nrow, 8)

    hs1_p, dis_p = _prep1_call(nrow)(x_p, degp, degp, expand, k1)

    a1 = _scatter_call(n_pad, h1, e // LANES)(
        hs1_p.reshape(n_pad, h1), ei2).reshape(2 * nrow, 128)

    hs2a_p, hs2b_p = _prep2_call(nrow)(a1, a1, hs1_p, dis_p, b1t, k2a, k2b)

    a2a = _scatter_call(n_pad, hh, e // LANES)(
        hs2a_p.reshape(n_pad, hh), ei2).reshape(2 * nrow, 128)
    a2b = _scatter_call(n_pad, hh, e // LANES)(
        hs2b_p.reshape(n_pad, hh), ei2).reshape(2 * nrow, 128)

    return _final_call(nrow, out_d)(
        a2a, a2a, a2b, a2b, hs2a_p, hs2b_p, dis_p,
        b2at, b2bt, bt, Wfc[:hh], Wfc[hh:], bfc.reshape(1, out_d))
